# baseline (speedup 1.0000x reference)
import functools

import jax
import jax.numpy as jnp
from jax import lax
from jax.experimental import pallas as pl
from jax.experimental.pallas import tpu as pltpu
from jax.experimental.pallas import tpu_sc as plsc

_G = 64
_NC = 2
_NS = 16
_NW = _NC * _NS
_GRP = 128
_K = 4
_CH = _K * _GRP
_NB = 2
_M0 = 32
_M1 = 8
_GW = (128, 32)


def _sc_segment_sum(xr, srclo2d, srchi2d, dst2d, zeros_stripe, n_pad, ep):
    half = xr.shape[1]
    rows_per_sub = n_pad // _NS
    q = ep // (_NS * _CH * (_M0 + _M1))
    m0, m1 = q * _M0, q * _M1
    rmax = m0 * _K

    mesh = plsc.VectorSubcoreMesh(core_axis_name="c", subcore_axis_name="s")

    @functools.partial(
        pl.kernel,
        out_type=jax.ShapeDtypeStruct((2, 2 * n_pad, half), jnp.float32),
        mesh=mesh,
        compiler_params=pltpu.CompilerParams(use_tc_tiling_on_sc=False),
        scratch_types=[
            pltpu.VMEM((rmax, _GRP), jnp.int32),
            pltpu.VMEM((_NB, _K, _GRP), jnp.int32),
            pltpu.VMEM((_NB, _CH, half), jnp.float32),
            pltpu.VMEM_SHARED((n_pad, half), jnp.float32),
            pltpu.SemaphoreType.DMA,
            pltpu.SemaphoreType.DMA,
            pltpu.SemaphoreType.DMA,
            pltpu.SemaphoreType.DMA,
        ],
    )
    def sc_agg(x_hbm, srclo_hbm, srchi_hbm, dst_hbm, z_hbm, out_hbm,
               src_v, dst_v, rows_v, agg_sh, gsem0, gsem1, dsem0, dsem1):
        c = lax.axis_index("c")
        s = lax.axis_index("s")
        rbase = s * rows_per_sub
        gsems = (gsem0, gsem1)
        dsems = (dsem0, dsem1)
        row0_c = (s * m0 * _K, (_NS * m0 + s * m1) * _K)
        nch_c = (m0, m1)

        def sweep(cc):
            row0 = row0_c[cc]
            gw = _GW[cc]
            nspr = _GRP // gw

            def gslices(g):
                out = []
                for j in range(_K):
                    for q in range(nspr):
                        idx = src_v.at[g * _K + j, pl.ds(q * gw, gw)]
                        out.append((idx, j * _GRP + q * gw))
                return out

            def fire(g, b):
                pltpu.async_copy(dst_hbm.at[pl.ds(row0 + g * _K, _K)],
                                 dst_v.at[b], dsems[b])
                for idx, off in gslices(g):
                    pltpu.async_copy(x_hbm.at[idx],
                                     rows_v.at[b, pl.ds(off, gw)],
                                     gsems[b])

            def drain_scatter(g, b):
                pltpu.make_async_copy(dst_hbm.at[pl.ds(row0 + g * _K, _K)],
                                      dst_v.at[b], dsems[b]).wait()
                for idx, off in gslices(g):
                    pltpu.make_async_copy(
                        x_hbm.at[idx],
                        rows_v.at[b, pl.ds(off, gw)],
                        gsems[b]).wait()
                for j in range(_K):
                    pltpu.sync_copy(rows_v.at[b, pl.ds(j * _GRP, _GRP)],
                                    agg_sh.at[dst_v.at[b, j]],
                                    add=True)

            nch = nch_c[cc]
            for b in range(_NB):
                fire(b, b)

            @pl.loop(_NB, nch, step=_NB)
            def _(g):
                for b in range(_NB):
                    drain_scatter(g + b - _NB, b)
                    fire(g + b, b)

            for b in range(_NB):
                drain_scatter(nch - _NB + b, b)

        for p, sidx_hbm in enumerate((srclo_hbm, srchi_hbm)):
            with jax.named_scope(f"init{p}"):
                pltpu.sync_copy(z_hbm, agg_sh.at[pl.ds(rbase, rows_per_sub)])
                for cc in range(_NC):
                    if nch_c[cc] == 0:
                        continue

                    @pl.when(c == cc)
                    def _():
                        nrows = nch_c[cc] * _K
                        pltpu.sync_copy(
                            sidx_hbm.at[pl.ds(row0_c[cc], nrows)],
                            src_v.at[pl.ds(0, nrows)])
                plsc.subcore_barrier()

            with jax.named_scope(f"sweep{p}"):
                for cc in range(_NC):
                    if nch_c[cc] == 0:
                        continue

                    @pl.when(c == cc)
                    def _():
                        sweep(cc)

                plsc.subcore_barrier()
            with jax.named_scope(f"wb{p}"):
                pltpu.sync_copy(agg_sh.at[pl.ds(rbase, rows_per_sub)],
                                out_hbm.at[p, pl.ds(c * n_pad + rbase,
                                                    rows_per_sub)])

    return sc_agg(xr, srclo2d, srchi2d, dst2d, zeros_stripe)


def _tc_gin_layer(x, parts, scale, W1, b1, W2, b2, gamma, beta, n_pad):
    n, d = x.shape

    def body(x_ref, p_ref, sc_ref, w1_ref, b1_ref, w2_ref, b2_ref,
             g_ref, be_ref, o_ref):
        agg = jnp.concatenate(
            [p_ref[0, 0:n, :] + p_ref[0, n_pad:n_pad + n, :],
             p_ref[1, 0:n, :] + p_ref[1, n_pad:n_pad + n, :]], axis=1)
        h = sc_ref[...] * x_ref[...] + agg
        t = jnp.maximum(
            jnp.dot(h, w1_ref[...], preferred_element_type=jnp.float32)
            + b1_ref[...], 0.0)
        h2 = (jnp.dot(t, w2_ref[...], preferred_element_type=jnp.float32)
              + b2_ref[...])
        mean = jnp.mean(h2, axis=0, keepdims=True)
        var = jnp.mean((h2 - mean) * (h2 - mean), axis=0, keepdims=True)
        hn = (h2 - mean) / jnp.sqrt(var + 1e-5) * g_ref[...] + be_ref[...]
        o_ref[...] = jnp.maximum(hn, 0.0)

    return pl.pallas_call(
        body,
        out_shape=jax.ShapeDtypeStruct((n, W2.shape[1]), jnp.float32),
    )(x, parts, scale, W1, b1, W2, b2, gamma, beta)


def _tc_final(h0, parts, batch2d, scale, W1, b1, W2, b2, gamma, beta,
              lin1_W, lin1_b, lin2_W, lin2_b, n_pad):
    n, d = h0.shape
    out_dim = lin2_W.shape[1]

    def body(x_ref, p_ref, seg_ref, sc_ref, w1_ref, b1_ref, w2_ref, b2_ref,
             g_ref, be_ref, l1w_ref, l1b_ref, l2w_ref, l2b_ref, o_ref):
        agg = jnp.concatenate(
            [p_ref[0, 0:n, :] + p_ref[0, n_pad:n_pad + n, :],
             p_ref[1, 0:n, :] + p_ref[1, n_pad:n_pad + n, :]], axis=1)
        h = sc_ref[...] * x_ref[...] + agg
        t = jnp.maximum(
            jnp.dot(h, w1_ref[...], preferred_element_type=jnp.float32)
            + b1_ref[...], 0.0)
        h2 = (jnp.dot(t, w2_ref[...], preferred_element_type=jnp.float32)
              + b2_ref[...])
        mean = jnp.mean(h2, axis=0, keepdims=True)
        var = jnp.mean((h2 - mean) * (h2 - mean), axis=0, keepdims=True)
        hn = (h2 - mean) / jnp.sqrt(var + 1e-5) * g_ref[...] + be_ref[...]
        h1 = jnp.maximum(hn, 0.0)
        ids = lax.broadcasted_iota(jnp.int32, (_G, n), 0)
        mask = (ids == seg_ref[...]).astype(jnp.float32)
        gsum = jnp.dot(mask, h1, preferred_element_type=jnp.float32)
        a = jnp.maximum(
            jnp.dot(gsum, l1w_ref[...], preferred_element_type=jnp.float32)
            + l1b_ref[...], 0.0)
        z = (jnp.dot(a, l2w_ref[...], preferred_element_type=jnp.float32)
             + l2b_ref[...])
        m = jnp.max(z, axis=-1, keepdims=True)
        lse = m + jnp.log(jnp.sum(jnp.exp(z - m), axis=-1, keepdims=True))
        o_ref[...] = z - lse

    return pl.pallas_call(
        body,
        out_shape=jax.ShapeDtypeStruct((_G, out_dim), jnp.float32),
    )(h0, parts, batch2d, scale, W1, b1, W2, b2, gamma, beta,
      lin1_W, lin1_b, lin2_W, lin2_b)


def kernel(x, edge_index, batch, W1_0, b1_0, W2_0, b2_0, eps_0, gamma_0,
           beta_0, W1_1, b1_1, W2_1, b2_1, eps_1, gamma_1, beta_1,
           lin1_W, lin1_b, lin2_W, lin2_b):
    n, d = x.shape
    e = edge_index.shape[1]

    rows_per_sub = -(-(n + 1) // _NS)
    rows_per_sub = -(-rows_per_sub // 8) * 8
    n_pad = rows_per_sub * _NS
    unit = _NS * _CH * (_M0 + _M1)
    ep = -(-e // unit) * unit

    pad = ep - e
    src_p = jnp.concatenate([edge_index[0], jnp.zeros((pad,), jnp.int32)])
    dst_p = jnp.concatenate([edge_index[1], jnp.full((pad,), n, jnp.int32)])
    srclo2d = (2 * src_p).reshape(ep // _GRP, _GRP)
    srchi2d = (2 * src_p + 1).reshape(ep // _GRP, _GRP)
    dst2d = dst_p.reshape(ep // _GRP, _GRP)
    zeros_stripe = jnp.zeros((rows_per_sub, d // 2), jnp.float32)
    batch2d = batch.reshape(1, n)

    scale0 = (1.0 + eps_0).reshape(1, 1)
    scale1 = (1.0 + eps_1).reshape(1, 1)
    b1_0r = b1_0.reshape(1, -1)
    b2_0r = b2_0.reshape(1, -1)
    b1_1r = b1_1.reshape(1, -1)
    b2_1r = b2_1.reshape(1, -1)
    gamma_0r = gamma_0.reshape(1, -1)
    beta_0r = beta_0.reshape(1, -1)
    gamma_1r = gamma_1.reshape(1, -1)
    beta_1r = beta_1.reshape(1, -1)
    lin1_br = lin1_b.reshape(1, -1)
    lin2_br = lin2_b.reshape(1, -1)

    p0 = _sc_segment_sum(x.reshape(2 * n, d // 2), srclo2d, srchi2d, dst2d,
                         zeros_stripe, n_pad, ep)
    h0 = _tc_gin_layer(x, p0, scale0, W1_0, b1_0r, W2_0, b2_0r,
                       gamma_0r, beta_0r, n_pad)
    p1 = _sc_segment_sum(h0.reshape(2 * n, d // 2), srclo2d, srchi2d, dst2d,
                         zeros_stripe, n_pad, ep)
    out = _tc_final(h0, p1, batch2d, scale1, W1_1, b1_1r, W2_1, b2_1r,
                    gamma_1r, beta_1r, lin1_W, lin1_br, lin2_W, lin2_br,
                    n_pad)
    return out


# speedup vs baseline: 1.1022x; 1.1022x over previous
import functools

import jax
import jax.numpy as jnp
from jax import lax
from jax.experimental import pallas as pl
from jax.experimental.pallas import tpu as pltpu
from jax.experimental.pallas import tpu_sc as plsc

_G = 64
_NC = 2
_NS = 16
_NW = _NC * _NS
_GRP = 128
_K = 4
_CH = _K * _GRP
_NB = 2
_M0 = 36
_M1 = 4
_GW = (128, 32)


def _sc_segment_sum(xr, srclo2d, srchi2d, dst2d, zeros_stripe, n_pad, ep):
    half = xr.shape[1]
    rows_per_sub = n_pad // _NS
    q = ep // (_NS * _CH * (_M0 + _M1))
    m0, m1 = q * _M0, q * _M1
    rmax = m0 * _K

    mesh = plsc.VectorSubcoreMesh(core_axis_name="c", subcore_axis_name="s")

    @functools.partial(
        pl.kernel,
        out_type=jax.ShapeDtypeStruct((2, 2 * n_pad, half), jnp.float32),
        mesh=mesh,
        compiler_params=pltpu.CompilerParams(use_tc_tiling_on_sc=False),
        scratch_types=[
            pltpu.VMEM((rmax, _GRP), jnp.int32),
            pltpu.VMEM((_NB, _K, _GRP), jnp.int32),
            pltpu.VMEM((_NB, _CH, half), jnp.float32),
            pltpu.VMEM_SHARED((n_pad, half), jnp.float32),
            pltpu.SemaphoreType.DMA,
            pltpu.SemaphoreType.DMA,
            pltpu.SemaphoreType.DMA,
            pltpu.SemaphoreType.DMA,
        ],
    )
    def sc_agg(x_hbm, srclo_hbm, srchi_hbm, dst_hbm, z_hbm, out_hbm,
               src_v, dst_v, rows_v, agg_sh, gsem0, gsem1, dsem0, dsem1):
        c = lax.axis_index("c")
        s = lax.axis_index("s")
        rbase = s * rows_per_sub
        gsems = (gsem0, gsem1)
        dsems = (dsem0, dsem1)
        row0_c = (s * m0 * _K, (_NS * m0 + s * m1) * _K)
        nch_c = (m0, m1)

        def sweep(cc):
            row0 = row0_c[cc]
            gw = _GW[cc]
            nspr = _GRP // gw

            def gslices(g):
                out = []
                for j in range(_K):
                    for q in range(nspr):
                        idx = src_v.at[g * _K + j, pl.ds(q * gw, gw)]
                        out.append((idx, j * _GRP + q * gw))
                return out

            def fire(g, b):
                pltpu.async_copy(dst_hbm.at[pl.ds(row0 + g * _K, _K)],
                                 dst_v.at[b], dsems[b])
                for idx, off in gslices(g):
                    pltpu.async_copy(x_hbm.at[idx],
                                     rows_v.at[b, pl.ds(off, gw)],
                                     gsems[b])

            def drain_scatter(g, b):
                pltpu.make_async_copy(dst_hbm.at[pl.ds(row0 + g * _K, _K)],
                                      dst_v.at[b], dsems[b]).wait()
                for idx, off in gslices(g):
                    pltpu.make_async_copy(
                        x_hbm.at[idx],
                        rows_v.at[b, pl.ds(off, gw)],
                        gsems[b]).wait()
                for j in range(_K):
                    pltpu.sync_copy(rows_v.at[b, pl.ds(j * _GRP, _GRP)],
                                    agg_sh.at[dst_v.at[b, j]],
                                    add=True)

            nch = nch_c[cc]
            for b in range(_NB):
                fire(b, b)

            @pl.loop(_NB, nch, step=_NB)
            def _(g):
                for b in range(_NB):
                    drain_scatter(g + b - _NB, b)
                    fire(g + b, b)

            for b in range(_NB):
                drain_scatter(nch - _NB + b, b)

        for p, sidx_hbm in enumerate((srclo_hbm, srchi_hbm)):
            with jax.named_scope(f"init{p}"):
                pltpu.sync_copy(z_hbm, agg_sh.at[pl.ds(rbase, rows_per_sub)])
                for cc in range(_NC):
                    if nch_c[cc] == 0:
                        continue

                    @pl.when(c == cc)
                    def _():
                        nrows = nch_c[cc] * _K
                        pltpu.sync_copy(
                            sidx_hbm.at[pl.ds(row0_c[cc], nrows)],
                            src_v.at[pl.ds(0, nrows)])
                plsc.subcore_barrier()

            with jax.named_scope(f"sweep{p}"):
                for cc in range(_NC):
                    if nch_c[cc] == 0:
                        continue

                    @pl.when(c == cc)
                    def _():
                        sweep(cc)

                plsc.subcore_barrier()
            with jax.named_scope(f"wb{p}"):
                pltpu.sync_copy(agg_sh.at[pl.ds(rbase, rows_per_sub)],
                                out_hbm.at[p, pl.ds(c * n_pad + rbase,
                                                    rows_per_sub)])

    return sc_agg(xr, srclo2d, srchi2d, dst2d, zeros_stripe)


def _tc_gin_layer(x, parts, scale, W1, b1, W2, b2, gamma, beta, n_pad):
    n, d = x.shape

    def body(x_ref, p_ref, sc_ref, w1_ref, b1_ref, w2_ref, b2_ref,
             g_ref, be_ref, o_ref):
        agg = jnp.concatenate(
            [p_ref[0, 0:n, :] + p_ref[0, n_pad:n_pad + n, :],
             p_ref[1, 0:n, :] + p_ref[1, n_pad:n_pad + n, :]], axis=1)
        h = sc_ref[...] * x_ref[...] + agg
        t = jnp.maximum(
            jnp.dot(h, w1_ref[...], preferred_element_type=jnp.float32)
            + b1_ref[...], 0.0)
        h2 = (jnp.dot(t, w2_ref[...], preferred_element_type=jnp.float32)
              + b2_ref[...])
        mean = jnp.mean(h2, axis=0, keepdims=True)
        var = jnp.mean((h2 - mean) * (h2 - mean), axis=0, keepdims=True)
        hn = (h2 - mean) / jnp.sqrt(var + 1e-5) * g_ref[...] + be_ref[...]
        o_ref[...] = jnp.maximum(hn, 0.0)

    return pl.pallas_call(
        body,
        out_shape=jax.ShapeDtypeStruct((n, W2.shape[1]), jnp.float32),
    )(x, parts, scale, W1, b1, W2, b2, gamma, beta)


def _tc_final(h0, parts, batch2d, scale, W1, b1, W2, b2, gamma, beta,
              lin1_W, lin1_b, lin2_W, lin2_b, n_pad):
    n, d = h0.shape
    out_dim = lin2_W.shape[1]

    def body(x_ref, p_ref, seg_ref, sc_ref, w1_ref, b1_ref, w2_ref, b2_ref,
             g_ref, be_ref, l1w_ref, l1b_ref, l2w_ref, l2b_ref, o_ref):
        agg = jnp.concatenate(
            [p_ref[0, 0:n, :] + p_ref[0, n_pad:n_pad + n, :],
             p_ref[1, 0:n, :] + p_ref[1, n_pad:n_pad + n, :]], axis=1)
        h = sc_ref[...] * x_ref[...] + agg
        t = jnp.maximum(
            jnp.dot(h, w1_ref[...], preferred_element_type=jnp.float32)
            + b1_ref[...], 0.0)
        h2 = (jnp.dot(t, w2_ref[...], preferred_element_type=jnp.float32)
              + b2_ref[...])
        mean = jnp.mean(h2, axis=0, keepdims=True)
        var = jnp.mean((h2 - mean) * (h2 - mean), axis=0, keepdims=True)
        hn = (h2 - mean) / jnp.sqrt(var + 1e-5) * g_ref[...] + be_ref[...]
        h1 = jnp.maximum(hn, 0.0)
        ids = lax.broadcasted_iota(jnp.int32, (_G, n), 0)
        mask = (ids == seg_ref[...]).astype(jnp.float32)
        gsum = jnp.dot(mask, h1, preferred_element_type=jnp.float32)
        a = jnp.maximum(
            jnp.dot(gsum, l1w_ref[...], preferred_element_type=jnp.float32)
            + l1b_ref[...], 0.0)
        z = (jnp.dot(a, l2w_ref[...], preferred_element_type=jnp.float32)
             + l2b_ref[...])
        m = jnp.max(z, axis=-1, keepdims=True)
        lse = m + jnp.log(jnp.sum(jnp.exp(z - m), axis=-1, keepdims=True))
        o_ref[...] = z - lse

    return pl.pallas_call(
        body,
        out_shape=jax.ShapeDtypeStruct((_G, out_dim), jnp.float32),
    )(h0, parts, batch2d, scale, W1, b1, W2, b2, gamma, beta,
      lin1_W, lin1_b, lin2_W, lin2_b)


def kernel(x, edge_index, batch, W1_0, b1_0, W2_0, b2_0, eps_0, gamma_0,
           beta_0, W1_1, b1_1, W2_1, b2_1, eps_1, gamma_1, beta_1,
           lin1_W, lin1_b, lin2_W, lin2_b):
    n, d = x.shape
    e = edge_index.shape[1]

    rows_per_sub = -(-(n + 1) // _NS)
    rows_per_sub = -(-rows_per_sub // 8) * 8
    n_pad = rows_per_sub * _NS
    unit = _NS * _CH * (_M0 + _M1)
    ep = -(-e // unit) * unit

    pad = ep - e
    src_p = jnp.concatenate([edge_index[0], jnp.zeros((pad,), jnp.int32)])
    dst_p = jnp.concatenate([edge_index[1], jnp.full((pad,), n, jnp.int32)])
    srclo2d = (2 * src_p).reshape(ep // _GRP, _GRP)
    srchi2d = (2 * src_p + 1).reshape(ep // _GRP, _GRP)
    dst2d = dst_p.reshape(ep // _GRP, _GRP)
    zeros_stripe = jnp.zeros((rows_per_sub, d // 2), jnp.float32)
    batch2d = batch.reshape(1, n)

    scale0 = (1.0 + eps_0).reshape(1, 1)
    scale1 = (1.0 + eps_1).reshape(1, 1)
    b1_0r = b1_0.reshape(1, -1)
    b2_0r = b2_0.reshape(1, -1)
    b1_1r = b1_1.reshape(1, -1)
    b2_1r = b2_1.reshape(1, -1)
    gamma_0r = gamma_0.reshape(1, -1)
    beta_0r = beta_0.reshape(1, -1)
    gamma_1r = gamma_1.reshape(1, -1)
    beta_1r = beta_1.reshape(1, -1)
    lin1_br = lin1_b.reshape(1, -1)
    lin2_br = lin2_b.reshape(1, -1)

    p0 = _sc_segment_sum(x.reshape(2 * n, d // 2), srclo2d, srchi2d, dst2d,
                         zeros_stripe, n_pad, ep)
    h0 = _tc_gin_layer(x, p0, scale0, W1_0, b1_0r, W2_0, b2_0r,
                       gamma_0r, beta_0r, n_pad)
    p1 = _sc_segment_sum(h0.reshape(2 * n, d // 2), srclo2d, srchi2d, dst2d,
                         zeros_stripe, n_pad, ep)
    out = _tc_final(h0, p1, batch2d, scale1, W1_1, b1_1r, W2_1, b2_1r,
                    gamma_1r, beta_1r, lin1_W, lin1_br, lin2_W, lin2_br,
                    n_pad)
    return out
